# speedup vs baseline: 33.9995x; 33.9995x over previous
import jax
import jax.numpy as jnp
from jax.experimental import pallas as pl
from jax.experimental.pallas import tpu as pltpu

_D = 512
_S = 16
_I = 1024
_C = 256
_B = 4
_L = 1024
_UNROLL = 8

_INTERPRET = False


def _front_kernel(x_ref, cw_ref, bng_ref, bnb_ref, bnm_ref, bnv_ref,
                  lng_ref, lnb_ref, inw_ref,
                  xssm_ref, gsil_ref, pool_ref):
    x = x_ref[0, 0]
    seq = jax.lax.dot_general(x, cw_ref[...], (((0,), (1,)), ((), ())),
                              preferred_element_type=jnp.float32)
    scale = bng_ref[...] * jax.lax.rsqrt(bnv_ref[...] + 1e-5)
    bias = bnb_ref[...] - bnm_ref[...] * scale
    seq = jnp.maximum(seq * scale + bias, 0.0)
    pool_ref[0, 0] = jnp.mean(seq, axis=0, keepdims=True)
    mu = jnp.mean(seq, axis=1, keepdims=True)
    dlt = seq - mu
    var = jnp.mean(dlt * dlt, axis=1, keepdims=True)
    xn = dlt * jax.lax.rsqrt(var + 1e-5) * lng_ref[0] + lnb_ref[0]
    xp = jax.lax.dot_general(xn, inw_ref[0], (((1,), (1,)), ((), ())),
                             preferred_element_type=jnp.float32)
    xssm_ref[0, 0] = xp[:, :_I]
    g = xp[:, _I:]
    gsil_ref[0, 0] = g * jax.nn.sigmoid(g)


def _lam_kernel(p_ref, w1_ref, b1_ref, w2_ref, b2_ref, lam_ref):
    p = p_ref[...]
    h1 = jax.lax.dot_general(p, w1_ref[...], (((1,), (1,)), ((), ())),
                             preferred_element_type=jnp.float32)
    h1 = jnp.maximum(h1 + b1_ref[...], 0.0)
    lg = jax.lax.dot_general(h1, w2_ref[...], (((1,), (1,)), ((), ())),
                             preferred_element_type=jnp.float32)
    lg = lg + b2_ref[...]
    m = jnp.max(lg, axis=1, keepdims=True)
    e = jnp.exp(lg - m)
    lam_ref[...] = e / jnp.sum(e, axis=1, keepdims=True)


def _scan_kernel(lam_ref, xssm_ref, gsil_ref, alog_ref, bmat_ref, cmat_ref,
                 gate_ref, outw_ref, resw_ref, rg_ref, rb_ref, rm_ref, rv_ref,
                 xorig_ref, out_ref, accf_ref, accb_ref):
    i = pl.program_id(0)
    b = jax.lax.rem(i, _B)
    A = jnp.clip(-jnp.exp(alog_ref[0]), -10.0, -0.01)
    Bm = bmat_ref[0]
    lam0 = lam_ref[b, 0]
    lam1 = lam_ref[b, 1]
    rowid = jax.lax.broadcasted_iota(jnp.int32, (2 * _S, _I), 0)
    lamv = jnp.where(rowid < _S, lam0, lam1)
    CL = cmat_ref[0] * lamv * 0.5

    n_tiles = _L // _UNROLL

    def make_body(reverse, acc_ref):
        def body(k, h):
            if reverse:
                base = pl.multiple_of((_L - _UNROLL) - k * _UNROLL, _UNROLL)
            else:
                base = pl.multiple_of(k * _UNROLL, _UNROLL)
            xt = xssm_ref[0, 0, pl.ds(base, _UNROLL), :]
            ys = [None] * _UNROLL
            for u in range(_UNROLL):
                j = (_UNROLL - 1 - u) if reverse else u
                xb = jnp.broadcast_to(xt[j:j + 1, :], (2 * _S, _I))
                h = jnp.clip(h * A + xb * Bm, -10.0, 10.0)
                ys[j] = jnp.sum(h * CL, axis=0, keepdims=True)
            acc_ref[pl.ds(base, _UNROLL), :] = jnp.concatenate(ys, axis=0)
            return h
        return body

    h0 = jnp.zeros((2 * _S, _I), jnp.float32)
    jax.lax.fori_loop(0, n_tiles, make_body(False, accf_ref), h0)
    jax.lax.fori_loop(0, n_tiles, make_body(True, accb_ref), h0)

    z = (accf_ref[...] + accb_ref[...]) * gsil_ref[0, 0]
    fT = jax.lax.dot_general(outw_ref[0], z, (((1,), (1,)), ((), ())),
                             preferred_element_type=jnp.float32)
    r = jax.lax.dot_general(resw_ref[...], fT, (((1,), (0,)), ((), ())),
                            preferred_element_type=jnp.float32)
    scale = rg_ref[...] * jax.lax.rsqrt(rv_ref[...] + 1e-5)
    bias = rb_ref[...] - rm_ref[...] * scale
    gsig = jax.nn.sigmoid(gate_ref[...])
    out_ref[0, 0] = xorig_ref[0, 0] + gsig * (r * scale + bias)


def kernel(x_V, x_I, conv_red_w, bn_red_g, bn_red_b, bn_red_m, bn_red_v,
           conv_res_w, bn_res_g, bn_res_b, bn_res_m, bn_res_v,
           lam_w1, lam_b1, lam_w2, lam_b2,
           V_in_w, V_out_w, V_A_log, V_B, V_C, V_ln_g, V_ln_b,
           I_in_w, I_out_w, I_A_log, I_B, I_C, I_ln_g, I_ln_b, gate):
    f32 = jnp.float32
    xs = jnp.stack([x_V.reshape(_B, _C, _L), x_I.reshape(_B, _C, _L)])
    ln_g2 = jnp.stack([V_ln_g, I_ln_g]).reshape(2, 1, _D)
    ln_b2 = jnp.stack([V_ln_b, I_ln_b]).reshape(2, 1, _D)
    in_w2 = jnp.stack([V_in_w, I_in_w])
    out_w2 = jnp.stack([V_out_w, I_out_w])
    alog2 = jnp.stack([jnp.concatenate([V_A_log.T, V_A_log.T], axis=0),
                       jnp.concatenate([I_A_log.T, I_A_log.T], axis=0)])
    bmat2 = jnp.stack([jnp.concatenate([V_B.T, I_B.T], axis=0),
                       jnp.concatenate([I_B.T, V_B.T], axis=0)])
    cmat2 = jnp.stack([jnp.concatenate([V_C.T, V_C.T], axis=0),
                       jnp.concatenate([I_C.T, I_C.T], axis=0)])

    bn2 = lambda v: v.reshape(1, _D)
    grid8 = (2 * _B,)
    sb = lambda i: (i // _B, i % _B, 0, 0)
    st = lambda i: (i // _B, 0, 0)
    whole2 = lambda i: (0, 0)

    xssm, gsil, pool = pl.pallas_call(
        _front_kernel,
        grid=grid8,
        in_specs=[
            pl.BlockSpec((1, 1, _C, _L), sb),
            pl.BlockSpec((_D, _C), whole2),
            pl.BlockSpec((1, _D), whole2),
            pl.BlockSpec((1, _D), whole2),
            pl.BlockSpec((1, _D), whole2),
            pl.BlockSpec((1, _D), whole2),
            pl.BlockSpec((1, 1, _D), st),
            pl.BlockSpec((1, 1, _D), st),
            pl.BlockSpec((1, 2 * _I, _D), st),
        ],
        out_specs=[
            pl.BlockSpec((1, 1, _L, _I), sb),
            pl.BlockSpec((1, 1, _L, _I), sb),
            pl.BlockSpec((1, 1, 1, _D), sb),
        ],
        out_shape=[
            jax.ShapeDtypeStruct((2, _B, _L, _I), f32),
            jax.ShapeDtypeStruct((2, _B, _L, _I), f32),
            jax.ShapeDtypeStruct((2, _B, 1, _D), f32),
        ],
        compiler_params=pltpu.CompilerParams(
            dimension_semantics=("parallel",),
        ),
        name="mamba_front",
        interpret=_INTERPRET,
    )(xs, conv_red_w, bn2(bn_red_g), bn2(bn_red_b), bn2(bn_red_m),
      bn2(bn_red_v), ln_g2, ln_b2, in_w2)

    p = jnp.concatenate([pool[0, :, 0, :], pool[1, :, 0, :]], axis=1)

    lam = pl.pallas_call(
        _lam_kernel,
        out_shape=jax.ShapeDtypeStruct((_B, 2), f32),
        name="mamba_lam",
        interpret=_INTERPRET,
    )(p, lam_w1, lam_b1.reshape(1, -1), lam_w2, lam_b2.reshape(1, -1))

    bnr = lambda v: v.reshape(_C, 1)
    out = pl.pallas_call(
        _scan_kernel,
        grid=grid8,
        in_specs=[
            pl.BlockSpec(memory_space=pltpu.SMEM),
            pl.BlockSpec((1, 1, _L, _I), sb),
            pl.BlockSpec((1, 1, _L, _I), sb),
            pl.BlockSpec((1, 2 * _S, _I), st),
            pl.BlockSpec((1, 2 * _S, _I), st),
            pl.BlockSpec((1, 2 * _S, _I), st),
            pl.BlockSpec((1, 1), whole2),
            pl.BlockSpec((1, _D, _I), st),
            pl.BlockSpec((_C, _D), whole2),
            pl.BlockSpec((_C, 1), whole2),
            pl.BlockSpec((_C, 1), whole2),
            pl.BlockSpec((_C, 1), whole2),
            pl.BlockSpec((_C, 1), whole2),
            pl.BlockSpec((1, 1, _C, _L), sb),
        ],
        out_specs=pl.BlockSpec((1, 1, _C, _L), sb),
        out_shape=jax.ShapeDtypeStruct((2, _B, _C, _L), f32),
        scratch_shapes=[
            pltpu.VMEM((_L, _I), f32),
            pltpu.VMEM((_L, _I), f32),
        ],
        compiler_params=pltpu.CompilerParams(
            dimension_semantics=("parallel",),
        ),
        name="mamba_scan",
        interpret=_INTERPRET,
    )(lam, xssm, gsil, alog2, bmat2, cmat2, gate.reshape(1, 1), out_w2,
      conv_res_w, bnr(bn_res_g), bnr(bn_res_b), bnr(bn_res_m), bnr(bn_res_v),
      xs)

    return (out[0].reshape(_B, _C, 32, 32), out[1].reshape(_B, _C, 32, 32))
